# baseline (device time: 54302 ns/iter reference)
import numpy as np
import jax
import jax.numpy as jnp
from jax import lax
from jax.experimental import pallas as pl
from jax.experimental.pallas import tpu as pltpu

N_DEV = 4
DH = 64


def kernel(x, Wq, Wk, Wv, Wo):
    B, s_loc, D = x.shape
    S = s_loc * N_DEV
    hd = Wq.shape[1]
    h_loc = hd // DH

    inv = 1.0 / (10000.0 ** (np.arange(0, DH, 2) / DH))
    pos = np.arange(S)[:, None] * inv[None, :]
    cos = np.repeat(np.cos(pos), 2, axis=-1).astype(np.float32)
    sin = np.repeat(np.sin(pos), 2, axis=-1).astype(np.float32)
    cosb = jnp.asarray(np.tile(cos, (1, h_loc)))
    sinb = jnp.asarray(np.tile(sin, (1, h_loc)))
    m = np.zeros((DH, DH), np.float32)
    for k in range(DH // 2):
        m[2 * k + 1, 2 * k] = -1.0
        m[2 * k, 2 * k + 1] = 1.0
    rotm = jnp.asarray(np.kron(np.eye(h_loc, dtype=np.float32), m))

    def body(x_ref, wq_ref, wk_ref, wv_ref, wo_ref, cos_ref, sin_ref, rot_ref,
             out_ref, xg_ref, partial_ref, rs_ref,
             ag_send, ag_recv, rs_send, rs_recv):
        my = lax.axis_index("i")
        left = lax.rem(my + N_DEV - 1, N_DEV)
        right = lax.rem(my + 1, N_DEV)

        barrier = pltpu.get_barrier_semaphore()
        for nbr in (left, right):
            pl.semaphore_signal(barrier, inc=1, device_id=(nbr,),
                                device_id_type=pl.DeviceIdType.MESH)
        pl.semaphore_wait(barrier, 2)

        xg_ref[:, pl.ds(my * s_loc, s_loc), :] = x_ref[...].astype(jnp.bfloat16)
        for h in range(N_DEV - 1):
            src_c = lax.rem(my - h + N_DEV, N_DEV)
            rdma = pltpu.make_async_remote_copy(
                src_ref=xg_ref.at[:, pl.ds(src_c * s_loc, s_loc), :],
                dst_ref=xg_ref.at[:, pl.ds(src_c * s_loc, s_loc), :],
                send_sem=ag_send.at[h],
                recv_sem=ag_recv.at[h],
                device_id=(right,),
                device_id_type=pl.DeviceIdType.MESH,
            )
            rdma.start()
            rdma.wait()

        wq = wq_ref[...].astype(jnp.bfloat16)
        wk = wk_ref[...].astype(jnp.bfloat16)
        wv = wv_ref[...].astype(jnp.bfloat16)
        wo = wo_ref[...].astype(jnp.bfloat16)
        cosv = cos_ref[...]
        sinv = sin_ref[...]
        rotv = rot_ref[...]
        for b in range(B):
            xb = xg_ref[b]
            qp = jnp.dot(xb, wq, preferred_element_type=jnp.float32)
            kp = jnp.dot(xb, wk, preferred_element_type=jnp.float32)
            vp = jnp.dot(xb, wv, preferred_element_type=jnp.float32)
            q = qp * cosv + jnp.dot(qp, rotv, preferred_element_type=jnp.float32) * sinv
            k = kp * cosv + jnp.dot(kp, rotv, preferred_element_type=jnp.float32) * sinv
            q = q.astype(jnp.bfloat16)
            k = k.astype(jnp.bfloat16)
            v = vp.astype(jnp.bfloat16)
            ctx_parts = []
            for h in range(h_loc):
                qh = q[:, h * DH:(h + 1) * DH]
                kh = k[:, h * DH:(h + 1) * DH]
                vh = v[:, h * DH:(h + 1) * DH]
                s = lax.dot_general(
                    qh, kh, (((1,), (1,)), ((), ())),
                    preferred_element_type=jnp.float32,
                ) * 0.125
                mx = jnp.max(s, axis=-1, keepdims=True)
                w = jnp.exp(s - mx)
                w = w / jnp.sum(w, axis=-1, keepdims=True)
                ctx_parts.append(jnp.dot(w.astype(jnp.bfloat16), vh,
                                         preferred_element_type=jnp.float32))
            ctx = jnp.concatenate(ctx_parts, axis=-1).astype(jnp.bfloat16)
            partial_ref[b] = jnp.dot(ctx, wo, preferred_element_type=jnp.float32)

        c0 = lax.rem(my - 1 + N_DEV, N_DEV)
        rs_ref[3] = partial_ref[:, pl.ds(c0 * s_loc, s_loc), :]
        send_slot = 3
        for t in range(N_DEV - 1):
            rdma = pltpu.make_async_remote_copy(
                src_ref=rs_ref.at[send_slot],
                dst_ref=rs_ref.at[t],
                send_sem=rs_send.at[t],
                recv_sem=rs_recv.at[t],
                device_id=(right,),
                device_id_type=pl.DeviceIdType.MESH,
            )
            rdma.start()
            rdma.wait()
            c = lax.rem(my - 2 - t + 2 * N_DEV, N_DEV)
            mine = partial_ref[:, pl.ds(c * s_loc, s_loc), :]
            if t < N_DEV - 2:
                rs_ref[t] = rs_ref[t] + mine
            else:
                out_ref[...] = rs_ref[t] + mine
            send_slot = t

    return pl.pallas_call(
        body,
        out_shape=jax.ShapeDtypeStruct((B, s_loc, D), jnp.float32),
        in_specs=[pl.BlockSpec(memory_space=pltpu.VMEM)] * 8,
        out_specs=pl.BlockSpec(memory_space=pltpu.VMEM),
        scratch_shapes=[
            pltpu.VMEM((B, S, D), jnp.bfloat16),
            pltpu.VMEM((B, S, D), jnp.float32),
            pltpu.VMEM((N_DEV, B, s_loc, D), jnp.float32),
            pltpu.SemaphoreType.DMA((N_DEV - 1,)),
            pltpu.SemaphoreType.DMA((N_DEV - 1,)),
            pltpu.SemaphoreType.DMA((N_DEV - 1,)),
            pltpu.SemaphoreType.DMA((N_DEV - 1,)),
        ],
        compiler_params=pltpu.CompilerParams(collective_id=0),
    )(x, Wq, Wk, Wv, Wo, cosb, sinb, rotm)


# device time: 43794 ns/iter; 1.2399x vs baseline; 1.2399x over previous
import numpy as np
import jax
import jax.numpy as jnp
from jax import lax
from jax.experimental import pallas as pl
from jax.experimental.pallas import tpu as pltpu

N_DEV = 4
DH = 64


def kernel(x, Wq, Wk, Wv, Wo):
    B, s_loc, D = x.shape
    S = s_loc * N_DEV
    hd = Wq.shape[1]
    h_loc = hd // DH

    inv = 1.0 / (10000.0 ** (np.arange(0, DH, 2) / DH))
    pos = np.arange(S)[:, None] * inv[None, :]
    cos = np.repeat(np.cos(pos), 2, axis=-1).astype(np.float32)
    sin = np.repeat(np.sin(pos), 2, axis=-1).astype(np.float32)
    cosb = jnp.asarray(np.tile(cos, (1, h_loc)))
    sinb = jnp.asarray(np.tile(sin, (1, h_loc)))
    m = np.zeros((DH, DH), np.float32)
    for k in range(DH // 2):
        m[2 * k + 1, 2 * k] = -1.0
        m[2 * k, 2 * k + 1] = 1.0
    rotm = jnp.asarray(np.kron(np.eye(h_loc, dtype=np.float32), m))

    def body(x_ref, wq_ref, wk_ref, wv_ref, wo_ref, cos_ref, sin_ref, rot_ref,
             out_ref, xg_ref, q_ref, k_ref, v_ref, rs_ref,
             ag_send, ag_recv, rs_send, rs_recv):
        my = lax.axis_index("i")
        left = lax.rem(my + N_DEV - 1, N_DEV)
        right = lax.rem(my + 1, N_DEV)

        barrier = pltpu.get_barrier_semaphore()
        for nbr in (left, right):
            pl.semaphore_signal(barrier, inc=1, device_id=(nbr,),
                                device_id_type=pl.DeviceIdType.MESH)
        pl.semaphore_wait(barrier, 2)

        wq = wq_ref[...].astype(jnp.bfloat16)
        wk = wk_ref[...].astype(jnp.bfloat16)
        wv = wv_ref[...].astype(jnp.bfloat16)
        wo = wo_ref[...].astype(jnp.bfloat16)
        rotv = rot_ref[...]

        def project_chunk(c):
            cosc = cos_ref[pl.ds(c * s_loc, s_loc), :]
            sinc = sin_ref[pl.ds(c * s_loc, s_loc), :]
            for b in range(B):
                xb = xg_ref[b, pl.ds(c * s_loc, s_loc), :]
                qp = jnp.dot(xb, wq, preferred_element_type=jnp.float32)
                kp = jnp.dot(xb, wk, preferred_element_type=jnp.float32)
                vp = jnp.dot(xb, wv, preferred_element_type=jnp.float32)
                q = qp * cosc + jnp.dot(qp, rotv,
                                        preferred_element_type=jnp.float32) * sinc
                k = kp * cosc + jnp.dot(kp, rotv,
                                        preferred_element_type=jnp.float32) * sinc
                q_ref[b, pl.ds(c * s_loc, s_loc), :] = q.astype(jnp.bfloat16)
                k_ref[b, pl.ds(c * s_loc, s_loc), :] = k.astype(jnp.bfloat16)
                v_ref[b, pl.ds(c * s_loc, s_loc), :] = vp.astype(jnp.bfloat16)

        xg_ref[:, pl.ds(my * s_loc, s_loc), :] = x_ref[...].astype(jnp.bfloat16)
        ag = []
        for h in range(N_DEV - 1):
            src_c = lax.rem(my - h + N_DEV, N_DEV)
            ag.append(pltpu.make_async_remote_copy(
                src_ref=xg_ref.at[:, pl.ds(src_c * s_loc, s_loc), :],
                dst_ref=xg_ref.at[:, pl.ds(src_c * s_loc, s_loc), :],
                send_sem=ag_send.at[h],
                recv_sem=ag_recv.at[h],
                device_id=(right,),
                device_id_type=pl.DeviceIdType.MESH,
            ))
        ag[0].start()
        project_chunk(my)
        for h in range(N_DEV - 1):
            ag[h].wait_recv()
            if h + 1 < N_DEV - 1:
                ag[h + 1].start()
            project_chunk(lax.rem(my - 1 - h + N_DEV, N_DEV))
            ag[h].wait_send()

        def attn_chunk(c):
            out = []
            for b in range(B):
                qb = q_ref[b, pl.ds(c * s_loc, s_loc), :]
                ctx_parts = []
                for h in range(h_loc):
                    qh = qb[:, h * DH:(h + 1) * DH]
                    kh = k_ref[b, :, h * DH:(h + 1) * DH]
                    vh = v_ref[b, :, h * DH:(h + 1) * DH]
                    s = lax.dot_general(
                        qh, kh, (((1,), (1,)), ((), ())),
                        preferred_element_type=jnp.float32,
                    ) * 0.125
                    mx = jnp.max(s, axis=-1, keepdims=True)
                    w = jnp.exp(s - mx)
                    w = w / jnp.sum(w, axis=-1, keepdims=True)
                    ctx_parts.append(jnp.dot(w.astype(jnp.bfloat16), vh,
                                             preferred_element_type=jnp.float32))
                ctx = jnp.concatenate(ctx_parts, axis=-1).astype(jnp.bfloat16)
                out.append(jnp.dot(ctx, wo, preferred_element_type=jnp.float32))
            return out

        rs = []
        for t in range(N_DEV - 1):
            rs.append(pltpu.make_async_remote_copy(
                src_ref=rs_ref.at[3 if t == 0 else t - 1],
                dst_ref=rs_ref.at[t],
                send_sem=rs_send.at[t],
                recv_sem=rs_recv.at[t],
                device_id=(right,),
                device_id_type=pl.DeviceIdType.MESH,
            ))

        for step in range(N_DEV):
            c = lax.rem(my - 1 - step + 2 * N_DEV, N_DEV)
            pc = attn_chunk(c)
            if step == 0:
                for b in range(B):
                    rs_ref[3, b] = pc[b].astype(jnp.bfloat16)
                rs[0].start()
            elif step < N_DEV - 1:
                rs[step - 1].wait_recv()
                for b in range(B):
                    rs_ref[step - 1, b] = (
                        rs_ref[step - 1, b].astype(jnp.float32) + pc[b]
                    ).astype(jnp.bfloat16)
                rs[step].start()
                rs[step - 1].wait_send()
            else:
                rs[step - 1].wait_recv()
                for b in range(B):
                    out_ref[b] = rs_ref[step - 1, b].astype(jnp.float32) + pc[b]
                rs[step - 1].wait_send()

    return pl.pallas_call(
        body,
        out_shape=jax.ShapeDtypeStruct((B, s_loc, D), jnp.float32),
        in_specs=[pl.BlockSpec(memory_space=pltpu.VMEM)] * 8,
        out_specs=pl.BlockSpec(memory_space=pltpu.VMEM),
        scratch_shapes=[
            pltpu.VMEM((B, S, D), jnp.bfloat16),
            pltpu.VMEM((B, S, hd), jnp.bfloat16),
            pltpu.VMEM((B, S, hd), jnp.bfloat16),
            pltpu.VMEM((B, S, hd), jnp.bfloat16),
            pltpu.VMEM((N_DEV, B, s_loc, D), jnp.bfloat16),
            pltpu.SemaphoreType.DMA((N_DEV - 1,)),
            pltpu.SemaphoreType.DMA((N_DEV - 1,)),
            pltpu.SemaphoreType.DMA((N_DEV - 1,)),
            pltpu.SemaphoreType.DMA((N_DEV - 1,)),
        ],
        compiler_params=pltpu.CompilerParams(collective_id=0),
    )(x, Wq, Wk, Wv, Wo, cosb, sinb, rotm)


# device time: 38572 ns/iter; 1.4078x vs baseline; 1.1354x over previous
import numpy as np
import jax
import jax.numpy as jnp
from jax import lax
from jax.experimental import pallas as pl
from jax.experimental.pallas import tpu as pltpu

N_DEV = 4
DH = 64


def kernel(x, Wq, Wk, Wv, Wo):
    B, s_loc, D = x.shape
    S = s_loc * N_DEV
    hd = Wq.shape[1]
    h_loc = hd // DH

    inv = 1.0 / (10000.0 ** (np.arange(0, DH, 2) / DH))
    pos = np.arange(S)[:, None] * inv[None, :]
    cos = np.repeat(np.cos(pos), 2, axis=-1).astype(np.float32)
    sin = np.repeat(np.sin(pos), 2, axis=-1).astype(np.float32)
    cosb = jnp.asarray(np.tile(cos, (1, h_loc)))
    sinb = jnp.asarray(np.tile(sin, (1, h_loc)))
    m = np.zeros((DH, DH), np.float32)
    for k in range(DH // 2):
        m[2 * k + 1, 2 * k] = -1.0
        m[2 * k, 2 * k + 1] = 1.0
    rotm = jnp.asarray(np.kron(np.eye(h_loc, dtype=np.float32), m))

    def body(x_ref, wq_ref, wk_ref, wv_ref, wo_ref, cos_ref, sin_ref, rot_ref,
             out_ref, xg_ref, q_ref, k_ref, v_ref, rs_ref,
             ag_send, ag_recv, rs_send, rs_recv):
        my = lax.axis_index("i")
        left = lax.rem(my + N_DEV - 1, N_DEV)
        right = lax.rem(my + 1, N_DEV)

        barrier = pltpu.get_barrier_semaphore()
        for nbr in (left, right):
            pl.semaphore_signal(barrier, inc=1, device_id=(nbr,),
                                device_id_type=pl.DeviceIdType.MESH)
        pl.semaphore_wait(barrier, 2)

        wq = wq_ref[...].astype(jnp.bfloat16)
        wk = wk_ref[...].astype(jnp.bfloat16)
        wv = wv_ref[...].astype(jnp.bfloat16)
        wo = wo_ref[...].astype(jnp.bfloat16)
        rotv = rot_ref[...]

        def project_chunk(c):
            cosc = cos_ref[pl.ds(c * s_loc, s_loc), :]
            sinc = sin_ref[pl.ds(c * s_loc, s_loc), :]
            for b in range(B):
                xb = xg_ref[b, pl.ds(c * s_loc, s_loc), :]
                qp = jnp.dot(xb, wq, preferred_element_type=jnp.float32)
                kp = jnp.dot(xb, wk, preferred_element_type=jnp.float32)
                vp = jnp.dot(xb, wv, preferred_element_type=jnp.float32)
                q = qp * cosc + jnp.dot(qp, rotv,
                                        preferred_element_type=jnp.float32) * sinc
                k = kp * cosc + jnp.dot(kp, rotv,
                                        preferred_element_type=jnp.float32) * sinc
                q_ref[b, pl.ds(c * s_loc, s_loc), :] = q.astype(jnp.bfloat16)
                k_ref[b, pl.ds(c * s_loc, s_loc), :] = k.astype(jnp.bfloat16)
                v_ref[b, pl.ds(c * s_loc, s_loc), :] = vp.astype(jnp.bfloat16)

        c_l = lax.rem(my - 1 + N_DEV, N_DEV)
        c_r = lax.rem(my + 1, N_DEV)
        c_d = lax.rem(my + 2, N_DEV)

        def ag_chunk_ref(c):
            return xg_ref.at[:, pl.ds(c * s_loc, s_loc), :]

        ag_A_r = pltpu.make_async_remote_copy(
            src_ref=ag_chunk_ref(my), dst_ref=ag_chunk_ref(my),
            send_sem=ag_send.at[0], recv_sem=ag_recv.at[0],
            device_id=(right,), device_id_type=pl.DeviceIdType.MESH)
        ag_A_l = pltpu.make_async_remote_copy(
            src_ref=ag_chunk_ref(my), dst_ref=ag_chunk_ref(my),
            send_sem=ag_send.at[1], recv_sem=ag_recv.at[1],
            device_id=(left,), device_id_type=pl.DeviceIdType.MESH)
        ag_B = pltpu.make_async_remote_copy(
            src_ref=ag_chunk_ref(c_l), dst_ref=ag_chunk_ref(c_l),
            send_sem=ag_send.at[2], recv_sem=ag_recv.at[2],
            device_id=(right,), device_id_type=pl.DeviceIdType.MESH)

        xg_ref[:, pl.ds(my * s_loc, s_loc), :] = x_ref[...].astype(jnp.bfloat16)
        ag_A_r.start()
        ag_A_l.start()
        project_chunk(my)
        ag_A_r.wait_recv()
        ag_B.start()
        project_chunk(c_l)
        ag_A_l.wait_recv()
        project_chunk(c_r)
        ag_B.wait_recv()
        project_chunk(c_d)
        ag_A_r.wait_send()
        ag_A_l.wait_send()
        ag_B.wait_send()

        def attn_chunk(c):
            out = []
            for b in range(B):
                qb = q_ref[b, pl.ds(c * s_loc, s_loc), :]
                ctx_parts = []
                for h in range(h_loc):
                    qh = qb[:, h * DH:(h + 1) * DH]
                    kh = k_ref[b, :, h * DH:(h + 1) * DH]
                    vh = v_ref[b, :, h * DH:(h + 1) * DH]
                    s = lax.dot_general(
                        qh, kh, (((1,), (1,)), ((), ())),
                        preferred_element_type=jnp.float32,
                    ) * 0.125
                    mx = jnp.max(s, axis=-1, keepdims=True)
                    w = jnp.exp(s - mx)
                    w = w / jnp.sum(w, axis=-1, keepdims=True)
                    ctx_parts.append(jnp.dot(w.astype(jnp.bfloat16), vh,
                                             preferred_element_type=jnp.float32))
                ctx = jnp.concatenate(ctx_parts, axis=-1).astype(jnp.bfloat16)
                out.append(jnp.dot(ctx, wo, preferred_element_type=jnp.float32))
            return out

        rs_A = pltpu.make_async_remote_copy(
            src_ref=rs_ref.at[0], dst_ref=rs_ref.at[3],
            send_sem=rs_send.at[0], recv_sem=rs_recv.at[0],
            device_id=(right,), device_id_type=pl.DeviceIdType.MESH)
        rs_B_r = pltpu.make_async_remote_copy(
            src_ref=rs_ref.at[1], dst_ref=rs_ref.at[4],
            send_sem=rs_send.at[1], recv_sem=rs_recv.at[1],
            device_id=(right,), device_id_type=pl.DeviceIdType.MESH)
        rs_B_l = pltpu.make_async_remote_copy(
            src_ref=rs_ref.at[2], dst_ref=rs_ref.at[5],
            send_sem=rs_send.at[2], recv_sem=rs_recv.at[2],
            device_id=(left,), device_id_type=pl.DeviceIdType.MESH)

        pc = attn_chunk(c_l)
        for b in range(B):
            rs_ref[2, b] = pc[b].astype(jnp.bfloat16)
        rs_B_l.start()
        pc = attn_chunk(c_d)
        for b in range(B):
            rs_ref[0, b] = pc[b].astype(jnp.bfloat16)
        rs_A.start()
        pc = attn_chunk(c_r)
        rs_A.wait_recv()
        for b in range(B):
            rs_ref[1, b] = (rs_ref[3, b].astype(jnp.float32)
                            + pc[b]).astype(jnp.bfloat16)
        rs_B_r.start()
        pc = attn_chunk(my)
        rs_B_r.wait_recv()
        rs_B_l.wait_recv()
        for b in range(B):
            out_ref[b] = (rs_ref[4, b].astype(jnp.float32)
                          + rs_ref[5, b].astype(jnp.float32) + pc[b])
        rs_A.wait_send()
        rs_B_r.wait_send()
        rs_B_l.wait_send()

    return pl.pallas_call(
        body,
        out_shape=jax.ShapeDtypeStruct((B, s_loc, D), jnp.float32),
        in_specs=[pl.BlockSpec(memory_space=pltpu.VMEM)] * 8,
        out_specs=pl.BlockSpec(memory_space=pltpu.VMEM),
        scratch_shapes=[
            pltpu.VMEM((B, S, D), jnp.bfloat16),
            pltpu.VMEM((B, S, hd), jnp.bfloat16),
            pltpu.VMEM((B, S, hd), jnp.bfloat16),
            pltpu.VMEM((B, S, hd), jnp.bfloat16),
            pltpu.VMEM((6, B, s_loc, D), jnp.bfloat16),
            pltpu.SemaphoreType.DMA((N_DEV - 1,)),
            pltpu.SemaphoreType.DMA((N_DEV - 1,)),
            pltpu.SemaphoreType.DMA((N_DEV - 1,)),
            pltpu.SemaphoreType.DMA((N_DEV - 1,)),
        ],
        compiler_params=pltpu.CompilerParams(collective_id=0),
    )(x, Wq, Wk, Wv, Wo, cosb, sinb, rotm)


# device time: 35605 ns/iter; 1.5251x vs baseline; 1.0833x over previous
import numpy as np
import jax
import jax.numpy as jnp
from jax import lax
from jax.experimental import pallas as pl
from jax.experimental.pallas import tpu as pltpu

N_DEV = 4
DH = 64


def kernel(x, Wq, Wk, Wv, Wo):
    B, s_loc, D = x.shape
    S = s_loc * N_DEV
    hd = Wq.shape[1]
    h_loc = hd // DH

    inv = 1.0 / (10000.0 ** (np.arange(0, DH, 2) / DH))
    pos = np.arange(S)[:, None] * inv[None, :]
    cos = np.repeat(np.cos(pos), 2, axis=-1).astype(np.float32)
    sin = np.repeat(np.sin(pos), 2, axis=-1).astype(np.float32)
    cosb = jnp.asarray(np.tile(cos, (1, h_loc)))
    sinb = jnp.asarray(np.tile(sin, (1, h_loc)))
    m = np.zeros((DH, DH), np.float32)
    for k in range(DH // 2):
        m[2 * k + 1, 2 * k] = -1.0
        m[2 * k, 2 * k + 1] = 1.0
    rotm = jnp.asarray(np.kron(np.eye(h_loc, dtype=np.float32), m))

    def body(x_ref, wq_ref, wk_ref, wv_ref, wo_ref, cos_ref, sin_ref, rot_ref,
             out_ref, xg_ref, q_ref, k_ref, v_ref, rs_ref,
             ag_send, ag_recv, rs_send, rs_recv):
        my = lax.axis_index("i")
        left = lax.rem(my + N_DEV - 1, N_DEV)
        right = lax.rem(my + 1, N_DEV)

        barrier = pltpu.get_barrier_semaphore()
        for nbr in (left, right):
            pl.semaphore_signal(barrier, inc=1, device_id=(nbr,),
                                device_id_type=pl.DeviceIdType.MESH)
        pl.semaphore_wait(barrier, 2)

        wq = wq_ref[...].astype(jnp.bfloat16)
        wk = wk_ref[...].astype(jnp.bfloat16)
        wv = wv_ref[...].astype(jnp.bfloat16)
        wo = wo_ref[...].astype(jnp.bfloat16)
        rotv = rot_ref[...]

        def project_chunk(c):
            cosc = cos_ref[pl.ds(c * s_loc, s_loc), :]
            sinc = sin_ref[pl.ds(c * s_loc, s_loc), :]
            for b in range(B):
                xb = xg_ref[b, pl.ds(c * s_loc, s_loc), :]
                qp = jnp.dot(xb, wq, preferred_element_type=jnp.float32)
                kp = jnp.dot(xb, wk, preferred_element_type=jnp.float32)
                vp = jnp.dot(xb, wv, preferred_element_type=jnp.float32)
                q = qp * cosc + jnp.dot(qp, rotv,
                                        preferred_element_type=jnp.float32) * sinc
                k = kp * cosc + jnp.dot(kp, rotv,
                                        preferred_element_type=jnp.float32) * sinc
                q_ref[b, pl.ds(c * s_loc, s_loc), :] = (q * 0.125).astype(jnp.bfloat16)
                k_ref[b, pl.ds(c * s_loc, s_loc), :] = k.astype(jnp.bfloat16)
                v_ref[b, pl.ds(c * s_loc, s_loc), :] = vp.astype(jnp.bfloat16)

        c_l = lax.rem(my - 1 + N_DEV, N_DEV)
        c_r = lax.rem(my + 1, N_DEV)
        c_d = lax.rem(my + 2, N_DEV)

        def ag_chunk_ref(c):
            return xg_ref.at[:, pl.ds(c * s_loc, s_loc), :]

        ag_A_r = pltpu.make_async_remote_copy(
            src_ref=ag_chunk_ref(my), dst_ref=ag_chunk_ref(my),
            send_sem=ag_send.at[0], recv_sem=ag_recv.at[0],
            device_id=(right,), device_id_type=pl.DeviceIdType.MESH)
        ag_A_l = pltpu.make_async_remote_copy(
            src_ref=ag_chunk_ref(my), dst_ref=ag_chunk_ref(my),
            send_sem=ag_send.at[1], recv_sem=ag_recv.at[1],
            device_id=(left,), device_id_type=pl.DeviceIdType.MESH)
        ag_B = pltpu.make_async_remote_copy(
            src_ref=ag_chunk_ref(c_l), dst_ref=ag_chunk_ref(c_l),
            send_sem=ag_send.at[2], recv_sem=ag_recv.at[2],
            device_id=(right,), device_id_type=pl.DeviceIdType.MESH)

        xg_ref[:, pl.ds(my * s_loc, s_loc), :] = x_ref[...].astype(jnp.bfloat16)
        ag_A_r.start()
        ag_A_l.start()
        project_chunk(my)
        ag_A_r.wait_recv()
        ag_B.start()
        project_chunk(c_l)
        ag_A_l.wait_recv()
        project_chunk(c_r)
        ag_B.wait_recv()
        project_chunk(c_d)
        ag_A_r.wait_send()
        ag_A_l.wait_send()
        ag_B.wait_send()

        def attn_chunk(c):
            out = []
            for b in range(B):
                qb = q_ref[b, pl.ds(c * s_loc, s_loc), :]
                ctx_parts = []
                for h in range(h_loc):
                    qh = qb[:, h * DH:(h + 1) * DH]
                    kh = k_ref[b, :, h * DH:(h + 1) * DH]
                    vh = v_ref[b, :, h * DH:(h + 1) * DH]
                    s = lax.dot_general(
                        qh, kh, (((1,), (1,)), ((), ())),
                        preferred_element_type=jnp.float32,
                    )
                    w = jnp.exp(s.astype(jnp.bfloat16))
                    denom = jnp.sum(w.astype(jnp.float32), axis=-1,
                                    keepdims=True)
                    ctx = jnp.dot(w, vh, preferred_element_type=jnp.float32)
                    ctx_parts.append(ctx / denom)
                ctx = jnp.concatenate(ctx_parts, axis=-1).astype(jnp.bfloat16)
                out.append(jnp.dot(ctx, wo, preferred_element_type=jnp.float32))
            return out

        rs_A = pltpu.make_async_remote_copy(
            src_ref=rs_ref.at[0], dst_ref=rs_ref.at[3],
            send_sem=rs_send.at[0], recv_sem=rs_recv.at[0],
            device_id=(right,), device_id_type=pl.DeviceIdType.MESH)
        rs_B_r = pltpu.make_async_remote_copy(
            src_ref=rs_ref.at[1], dst_ref=rs_ref.at[4],
            send_sem=rs_send.at[1], recv_sem=rs_recv.at[1],
            device_id=(right,), device_id_type=pl.DeviceIdType.MESH)
        rs_B_l = pltpu.make_async_remote_copy(
            src_ref=rs_ref.at[2], dst_ref=rs_ref.at[5],
            send_sem=rs_send.at[2], recv_sem=rs_recv.at[2],
            device_id=(left,), device_id_type=pl.DeviceIdType.MESH)

        pc = attn_chunk(c_l)
        for b in range(B):
            rs_ref[2, b] = pc[b].astype(jnp.bfloat16)
        rs_B_l.start()
        pc = attn_chunk(c_d)
        for b in range(B):
            rs_ref[0, b] = pc[b].astype(jnp.bfloat16)
        rs_A.start()
        pc = attn_chunk(c_r)
        rs_A.wait_recv()
        for b in range(B):
            rs_ref[1, b] = (rs_ref[3, b].astype(jnp.float32)
                            + pc[b]).astype(jnp.bfloat16)
        rs_B_r.start()
        pc = attn_chunk(my)
        rs_B_r.wait_recv()
        rs_B_l.wait_recv()
        for b in range(B):
            out_ref[b] = (rs_ref[4, b].astype(jnp.float32)
                          + rs_ref[5, b].astype(jnp.float32) + pc[b])
        rs_A.wait_send()
        rs_B_r.wait_send()
        rs_B_l.wait_send()

    return pl.pallas_call(
        body,
        out_shape=jax.ShapeDtypeStruct((B, s_loc, D), jnp.float32),
        in_specs=[pl.BlockSpec(memory_space=pltpu.VMEM)] * 8,
        out_specs=pl.BlockSpec(memory_space=pltpu.VMEM),
        scratch_shapes=[
            pltpu.VMEM((B, S, D), jnp.bfloat16),
            pltpu.VMEM((B, S, hd), jnp.bfloat16),
            pltpu.VMEM((B, S, hd), jnp.bfloat16),
            pltpu.VMEM((B, S, hd), jnp.bfloat16),
            pltpu.VMEM((6, B, s_loc, D), jnp.bfloat16),
            pltpu.SemaphoreType.DMA((N_DEV - 1,)),
            pltpu.SemaphoreType.DMA((N_DEV - 1,)),
            pltpu.SemaphoreType.DMA((N_DEV - 1,)),
            pltpu.SemaphoreType.DMA((N_DEV - 1,)),
        ],
        compiler_params=pltpu.CompilerParams(collective_id=0),
    )(x, Wq, Wk, Wv, Wo, cosb, sinb, rotm)


# device time: 22268 ns/iter; 2.4386x vs baseline; 1.5989x over previous
import numpy as np
import jax
import jax.numpy as jnp
from jax import lax
from jax.experimental import pallas as pl
from jax.experimental.pallas import tpu as pltpu

N_DEV = 4
DH = 64


def kernel(x, Wq, Wk, Wv, Wo):
    B, s_loc, D = x.shape
    S = s_loc * N_DEV
    hd = Wq.shape[1]
    h_loc = hd // DH
    s_half = s_loc // 2

    inv = 1.0 / (10000.0 ** (np.arange(0, DH, 2) / DH))
    pos = np.arange(S)[:, None] * inv[None, :]
    cos_t = jnp.asarray(np.repeat(np.cos(pos), 2, axis=-1).astype(np.float32))
    sin_t = jnp.asarray(np.repeat(np.sin(pos), 2, axis=-1).astype(np.float32))
    m = np.zeros((DH, DH), np.float32)
    for k in range(DH // 2):
        m[2 * k + 1, 2 * k] = -1.0
        m[2 * k, 2 * k + 1] = 1.0
    rotm = jnp.asarray(
        np.kron(np.eye(h_loc), m).astype(np.float32)).astype(jnp.bfloat16)

    def body(x_ref, wq_ref, wk_ref, wv_ref, wo_ref, cos_ref, sin_ref, rot_ref,
             out_ref, xg_ref, q_ref, k_ref, v_ref, rs_ref,
             x_v, wqkv_v, wo_v, cos_v, sin_v, rot_v,
             ag_send, ag_recv, rs_send, rs_recv, ld_sems):
        my = lax.axis_index("i")
        left = lax.rem(my + N_DEV - 1, N_DEV)
        right = lax.rem(my + 1, N_DEV)
        c_l = left
        c_r = right
        c_d = lax.rem(my + 2, N_DEV)

        loads = [
            pltpu.make_async_copy(src, dst, ld_sems.at[i])
            for i, (src, dst) in enumerate([
                (x_ref, x_v),
                (wq_ref, wqkv_v.at[:, pl.ds(0, hd)]),
                (wk_ref, wqkv_v.at[:, pl.ds(hd, hd)]),
                (wv_ref, wqkv_v.at[:, pl.ds(2 * hd, hd)]),
                (cos_ref, cos_v), (sin_ref, sin_v), (rot_ref, rot_v),
                (wo_ref, wo_v),
            ])
        ]
        for ld in loads:
            ld.start()

        barrier = pltpu.get_barrier_semaphore()
        for nbr in (left, right):
            pl.semaphore_signal(barrier, inc=1, device_id=(nbr,),
                                device_id_type=pl.DeviceIdType.MESH)

        def project_chunk(c):
            w = wqkv_v[...].astype(jnp.bfloat16)
            rotv = rot_v[...]
            cosc = jnp.tile(cos_v[pl.ds(c * s_loc, s_loc), :], (1, h_loc))
            sinc = jnp.tile(sin_v[pl.ds(c * s_loc, s_loc), :], (1, h_loc))
            for b in range(B):
                xb = xg_ref[b, pl.ds(c * s_loc, s_loc), :]
                qkv = jnp.dot(xb, w, preferred_element_type=jnp.float32)
                qp = qkv[:, :hd]
                kp = qkv[:, hd:2 * hd]
                qk_b = jnp.concatenate(
                    [qp, kp], axis=0).astype(jnp.bfloat16)
                qkr = jnp.dot(qk_b, rotv, preferred_element_type=jnp.float32)
                q = qp * cosc + qkr[:s_loc] * sinc
                k = kp * cosc + qkr[s_loc:] * sinc
                q_ref[b, pl.ds(c * s_loc, s_loc), :] = (q * 0.125).astype(jnp.bfloat16)
                k_ref[b, pl.ds(c * s_loc, s_loc), :] = k.astype(jnp.bfloat16)
                v_ref[b, pl.ds(c * s_loc, s_loc), :] = (
                    qkv[:, 2 * hd:].astype(jnp.bfloat16))

        def ag_rows(c, off, n):
            return xg_ref.at[:, pl.ds(c * s_loc + off, n), :]

        def ag_rdma(src_c, off, n, sem_i, dev):
            return pltpu.make_async_remote_copy(
                src_ref=ag_rows(src_c, off, n), dst_ref=ag_rows(src_c, off, n),
                send_sem=ag_send.at[sem_i], recv_sem=ag_recv.at[sem_i],
                device_id=(dev,), device_id_type=pl.DeviceIdType.MESH)

        ag_A_r1 = ag_rdma(my, 0, s_half, 0, right)
        ag_A_r2 = ag_rdma(my, s_half, s_half, 1, right)
        ag_A_l1 = ag_rdma(my, s_half, s_half, 2, left)
        ag_A_l2 = ag_rdma(my, 0, s_half, 3, left)
        ag_B_r = ag_rdma(c_l, 0, s_half, 4, right)
        ag_B_l = ag_rdma(c_r, s_half, s_half, 5, left)

        loads[0].wait()
        xg_ref[:, pl.ds(my * s_loc, s_loc), :] = x_v[...].astype(jnp.bfloat16)
        pl.semaphore_wait(barrier, 2)
        ag_A_r1.start()
        ag_A_l1.start()
        ag_A_r2.start()
        ag_A_l2.start()
        for ld in loads[1:7]:
            ld.wait()
        project_chunk(my)
        ag_A_r1.wait_recv()
        ag_B_r.start()
        ag_A_l1.wait_recv()
        ag_B_l.start()
        ag_A_r2.wait_recv()
        project_chunk(c_l)
        ag_A_l2.wait_recv()
        project_chunk(c_r)
        ag_B_r.wait_recv()
        ag_B_l.wait_recv()
        project_chunk(c_d)
        for d in (ag_A_r1, ag_A_r2, ag_A_l1, ag_A_l2, ag_B_r, ag_B_l):
            d.wait_send()

        def attn_chunk(c, off=0, n=None):
            n = s_loc if n is None else n
            wo = wo_v[...].astype(jnp.bfloat16)
            out = []
            for b in range(B):
                qb = q_ref[b, pl.ds(c * s_loc + off, n), :]
                ctx_parts = []
                for h in range(h_loc):
                    qh = qb[:, h * DH:(h + 1) * DH]
                    kh = k_ref[b, :, h * DH:(h + 1) * DH]
                    vh = v_ref[b, :, h * DH:(h + 1) * DH]
                    s = lax.dot_general(
                        qh, kh, (((1,), (1,)), ((), ())),
                        preferred_element_type=jnp.float32,
                    )
                    w = jnp.exp(s.astype(jnp.bfloat16))
                    denom = jnp.sum(w.astype(jnp.float32), axis=-1,
                                    keepdims=True)
                    ctx = jnp.dot(w, vh, preferred_element_type=jnp.float32)
                    ctx_parts.append(ctx / denom)
                ctx = jnp.concatenate(ctx_parts, axis=-1).astype(jnp.bfloat16)
                out.append(jnp.dot(ctx, wo, preferred_element_type=jnp.float32))
            return out

        def rs_rows(slot, off, n):
            return rs_ref.at[slot, :, pl.ds(off, n), :]

        rs_A_r = pltpu.make_async_remote_copy(
            src_ref=rs_rows(0, 0, s_half), dst_ref=rs_rows(3, 0, s_half),
            send_sem=rs_send.at[0], recv_sem=rs_recv.at[0],
            device_id=(right,), device_id_type=pl.DeviceIdType.MESH)
        rs_A_l = pltpu.make_async_remote_copy(
            src_ref=rs_rows(0, s_half, s_half),
            dst_ref=rs_rows(3, s_half, s_half),
            send_sem=rs_send.at[3], recv_sem=rs_recv.at[3],
            device_id=(left,), device_id_type=pl.DeviceIdType.MESH)
        rs_B_r = pltpu.make_async_remote_copy(
            src_ref=rs_ref.at[1], dst_ref=rs_ref.at[4],
            send_sem=rs_send.at[1], recv_sem=rs_recv.at[1],
            device_id=(right,), device_id_type=pl.DeviceIdType.MESH)
        rs_B_l = pltpu.make_async_remote_copy(
            src_ref=rs_ref.at[2], dst_ref=rs_ref.at[5],
            send_sem=rs_send.at[2], recv_sem=rs_recv.at[2],
            device_id=(left,), device_id_type=pl.DeviceIdType.MESH)

        loads[7].wait()
        pc = attn_chunk(c_d)
        for b in range(B):
            rs_ref[0, b] = pc[b].astype(jnp.bfloat16)
        rs_A_r.start()
        rs_A_l.start()
        pc_r = attn_chunk(c_r)
        pc_l = attn_chunk(c_l)
        rs_A_r.wait_recv()
        for b in range(B):
            rs_ref[1, b, :s_half] = (
                rs_ref[3, b, :s_half].astype(jnp.float32) + pc_r[b][:s_half]
            ).astype(jnp.bfloat16)
            rs_ref[1, b, s_half:] = pc_r[b][s_half:].astype(jnp.bfloat16)
        rs_B_r.start()
        rs_A_l.wait_recv()
        for b in range(B):
            rs_ref[2, b, :s_half] = pc_l[b][:s_half].astype(jnp.bfloat16)
            rs_ref[2, b, s_half:] = (
                rs_ref[3, b, s_half:].astype(jnp.float32) + pc_l[b][s_half:]
            ).astype(jnp.bfloat16)
        rs_B_l.start()
        pc = attn_chunk(my)
        rs_B_r.wait_recv()
        rs_B_l.wait_recv()
        for b in range(B):
            out_ref[b] = (rs_ref[4, b].astype(jnp.float32)
                          + rs_ref[5, b].astype(jnp.float32)
                          + pc[b]).astype(jnp.bfloat16)
        rs_A_r.wait_send()
        rs_A_l.wait_send()
        rs_B_r.wait_send()
        rs_B_l.wait_send()

    return pl.pallas_call(
        body,
        out_shape=jax.ShapeDtypeStruct((B, s_loc, D), jnp.bfloat16),
        in_specs=[pl.BlockSpec(memory_space=pltpu.MemorySpace.HBM)] * 8,
        out_specs=pl.BlockSpec(memory_space=pltpu.VMEM),
        scratch_shapes=[
            pltpu.VMEM((B, S, D), jnp.bfloat16),
            pltpu.VMEM((B, S, hd), jnp.bfloat16),
            pltpu.VMEM((B, S, hd), jnp.bfloat16),
            pltpu.VMEM((B, S, hd), jnp.bfloat16),
            pltpu.VMEM((6, B, s_loc, D), jnp.bfloat16),
            pltpu.VMEM((B, s_loc, D), jnp.float32),
            pltpu.VMEM((D, 3 * hd), jnp.float32),
            pltpu.VMEM((hd, D), jnp.float32),
            pltpu.VMEM((S, DH), jnp.float32),
            pltpu.VMEM((S, DH), jnp.float32),
            pltpu.VMEM((hd, hd), jnp.bfloat16),
            pltpu.SemaphoreType.DMA((6,)),
            pltpu.SemaphoreType.DMA((6,)),
            pltpu.SemaphoreType.DMA((4,)),
            pltpu.SemaphoreType.DMA((4,)),
            pltpu.SemaphoreType.DMA((8,)),
        ],
        compiler_params=pltpu.CompilerParams(collective_id=0),
    )(*[
        pltpu.with_memory_space_constraint(a, pltpu.MemorySpace.HBM)
        for a in (x, Wq, Wk, Wv, Wo, cos_t, sin_t, rotm)
    ])
